# Initial kernel scaffold; baseline (speedup 1.0000x reference)
import jax
import jax.numpy as jnp
from jax.experimental import pallas as pl


def kernel(x, edge_index, w, W1, b1, W2, b2, W3, b3, W4, b4, Wp, bp, Wv, bv):
    raise NotImplementedError("write your pallas kernel here")


# speedup vs baseline: 1.8713x; 1.8713x over previous
import functools

import jax
import jax.numpy as jnp
from jax import lax
from jax.experimental import pallas as pl
from jax.experimental.pallas import tpu as pltpu
from jax.experimental.pallas import tpu_sc as plsc

N_NODES = 10000
N_PAD = 10240
E = 160000
NC = 2
NS = 16
KB = 128
E_PER_TILE = 5120
E_PAD = E_PER_TILE * NC * NS
BATCHES = E_PER_TILE // KB
ROWS_PER_TILE = N_PAD // NS


@functools.lru_cache(maxsize=None)
def _make_sc_scatter(n_chunks: int, fc: int):
    mesh = plsc.VectorSubcoreMesh(core_axis_name="c", subcore_axis_name="s")
    n_lane = fc // 16

    @functools.partial(
        pl.kernel,
        out_type=jax.ShapeDtypeStruct((n_chunks, NC, N_PAD, fc), jnp.float32),
        mesh=mesh,
        scratch_types=[
            pltpu.VMEM((BATCHES, KB), jnp.int32),
            pltpu.VMEM((BATCHES, KB), jnp.int32),
            pltpu.VMEM((BATCHES, KB), jnp.float32),
            pltpu.VMEM((KB, fc), jnp.float32),
            pltpu.VMEM((KB, fc), jnp.float32),
            pltpu.VMEM_SHARED((N_PAD, fc), jnp.float32),
            pltpu.SemaphoreType.DMA,
        ],
    )
    def sc_scatter(*refs):
        tables = refs[:n_chunks]
        src_h, dst_h, w_h, out = refs[n_chunks:n_chunks + 4]
        src_v, dst_v, w_v, buf, zbuf, acc, sem = refs[n_chunks + 4:]

        cid = lax.axis_index("c")
        sid = lax.axis_index("s")
        wid = cid * NS + sid
        pltpu.sync_copy(src_h.at[wid], src_v)
        pltpu.sync_copy(dst_h.at[wid], dst_v)
        pltpu.sync_copy(w_h.at[wid], w_v)

        def zrow(i, carry):
            for l in range(n_lane):
                zbuf[i, pl.ds(l * 16, 16)] = jnp.zeros((16,), jnp.float32)
            return carry
        lax.fori_loop(0, KB, zrow, 0)

        row0 = sid * ROWS_PER_TILE
        for c in range(n_chunks):
            def zblk(b, carry):
                pltpu.sync_copy(zbuf, acc.at[pl.ds(row0 + b * KB, KB)])
                return carry
            lax.fori_loop(0, ROWS_PER_TILE // KB, zblk, 0)
            plsc.subcore_barrier()

            def batch(j, carry):
                pltpu.async_copy(tables[c].at[src_v.at[j]], buf, sem).wait()

                def scale(g, carry2):
                    wvec = w_v[j, pl.ds(g * 16, 16)]
                    for i2 in range(16):
                        s = wvec[i2]
                        r = g * 16 + i2
                        for l in range(n_lane):
                            buf[r, pl.ds(l * 16, 16)] = buf[r, pl.ds(l * 16, 16)] * s
                    return carry2
                lax.fori_loop(0, KB // 16, scale, 0)
                pltpu.sync_copy(buf, acc.at[dst_v.at[j]], add=True)
                return carry
            lax.fori_loop(0, BATCHES, batch, 0)
            plsc.subcore_barrier()

            pltpu.sync_copy(acc.at[pl.ds(row0, ROWS_PER_TILE)],
                            out.at[c, cid, pl.ds(row0, ROWS_PER_TILE)])

    return sc_scatter


def _mm_body(p_ref, w_ref, b_ref, o_ref, acc_ref, *, n_ci, relu):
    ci = pl.program_id(2)

    @pl.when(ci == 0)
    def _():
        acc_ref[...] = jnp.zeros_like(acc_ref)

    a = p_ref[0, 0] + p_ref[0, 1]
    acc_ref[...] += jnp.dot(a, w_ref[...], preferred_element_type=jnp.float32)

    @pl.when(ci == n_ci - 1)
    def _():
        r = acc_ref[...] + b_ref[...]
        if relu:
            r = jnp.maximum(r, 0.0)
        o_ref[0] = r


def _tc_matmul(p, w, b, relu, nb=1024):
    n_ci, _, n_pad, fc = p.shape
    n_co = w.shape[1] // fc
    grid = (n_pad // nb, n_co, n_ci)
    return pl.pallas_call(
        functools.partial(_mm_body, n_ci=n_ci, relu=relu),
        grid=grid,
        in_specs=[
            pl.BlockSpec((1, 2, nb, fc), lambda i, co, ci: (ci, 0, i, 0)),
            pl.BlockSpec((fc, fc), lambda i, co, ci: (ci, co)),
            pl.BlockSpec((1, fc), lambda i, co, ci: (0, co)),
        ],
        out_specs=pl.BlockSpec((1, nb, fc), lambda i, co, ci: (co, i, 0)),
        out_shape=jax.ShapeDtypeStruct((n_co, n_pad, fc), jnp.float32),
        scratch_shapes=[pltpu.VMEM((nb, fc), jnp.float32)],
    )(p, w, b.reshape(1, -1))


def _mm_fused_body(p_ref, w3_ref, b3_ref, w4_ref, o_ref, acc_ref, *, n_ci):
    ci = pl.program_id(1)

    @pl.when(ci == 0)
    def _():
        acc_ref[...] = jnp.zeros_like(acc_ref)

    a = p_ref[0, 0] + p_ref[0, 1]
    acc_ref[...] += jnp.dot(a, w3_ref[...], preferred_element_type=jnp.float32)

    @pl.when(ci == n_ci - 1)
    def _():
        h = jnp.maximum(acc_ref[...] + b3_ref[...], 0.0)
        o_ref[...] = jnp.dot(h, w4_ref[...], preferred_element_type=jnp.float32)


def _tc_matmul_fused(p, w3, b3, w4, nb=1024):
    n_ci, _, n_pad, fc = p.shape
    h = w3.shape[1]
    f_out = w4.shape[1]
    grid = (n_pad // nb, n_ci)
    return pl.pallas_call(
        functools.partial(_mm_fused_body, n_ci=n_ci),
        grid=grid,
        in_specs=[
            pl.BlockSpec((1, 2, nb, fc), lambda i, ci: (ci, 0, i, 0)),
            pl.BlockSpec((fc, h), lambda i, ci: (ci, 0)),
            pl.BlockSpec((1, h), lambda i, ci: (0, 0)),
            pl.BlockSpec((h, f_out), lambda i, ci: (0, 0)),
        ],
        out_specs=pl.BlockSpec((nb, f_out), lambda i, ci: (i, 0)),
        out_shape=jax.ShapeDtypeStruct((n_pad, f_out), jnp.float32),
        scratch_shapes=[pltpu.VMEM((nb, h), jnp.float32)],
    )(p, w3, b3.reshape(1, -1), w4)


def _readout_body(p_ref, b4_ref, wp_ref, bp_ref, wv_ref, bv_ref,
                  pi_ref, v_ref, cs_ref, *, nb, n_blocks):
    i = pl.program_id(0)
    h = p_ref[0, 0] + p_ref[0, 1] + b4_ref[...]
    rows = i * nb + lax.broadcasted_iota(jnp.int32, (nb, 1), 0)
    hm = jnp.where(rows < N_NODES, h, 0.0)
    pi_ref[...] = jnp.dot(hm, wp_ref[...], preferred_element_type=jnp.float32) + bp_ref[...]

    @pl.when(i == 0)
    def _():
        cs_ref[...] = jnp.zeros_like(cs_ref)

    cs_ref[...] += jnp.sum(hm, axis=0, keepdims=True)

    @pl.when(i == n_blocks - 1)
    def _():
        v_ref[...] = jnp.dot(cs_ref[...], wv_ref[...],
                             preferred_element_type=jnp.float32) + bv_ref[...]


def _tc_readout(p, b4, wp, bp, wv, bv, nb=1024):
    _, _, n_pad, fc = p.shape
    n_blocks = n_pad // nb
    return pl.pallas_call(
        functools.partial(_readout_body, nb=nb, n_blocks=n_blocks),
        grid=(n_blocks,),
        in_specs=[
            pl.BlockSpec((1, 2, nb, fc), lambda i: (0, 0, i, 0)),
            pl.BlockSpec((1, fc), lambda i: (0, 0)),
            pl.BlockSpec((fc, 1), lambda i: (0, 0)),
            pl.BlockSpec((1, 1), lambda i: (0, 0)),
            pl.BlockSpec((fc, 1), lambda i: (0, 0)),
            pl.BlockSpec((1, 1), lambda i: (0, 0)),
        ],
        out_specs=[
            pl.BlockSpec((nb, 1), lambda i: (i, 0)),
            pl.BlockSpec((1, 1), lambda i: (0, 0)),
        ],
        out_shape=[
            jax.ShapeDtypeStruct((n_pad, 1), jnp.float32),
            jax.ShapeDtypeStruct((1, 1), jnp.float32),
        ],
        scratch_shapes=[pltpu.VMEM((1, fc), jnp.float32)],
    )(p, b4.reshape(1, -1), wp, bp.reshape(1, 1), wv, bv.reshape(1, 1))


def kernel(x, edge_index, w, W1, b1, W2, b2, W3, b3, W4, b4, Wp, bp, Wv, bv):
    src = jnp.pad(edge_index[0].astype(jnp.int32), (0, E_PAD - E))
    dst = jnp.pad(edge_index[1].astype(jnp.int32), (0, E_PAD - E))
    wgt = jnp.pad(w.astype(jnp.float32), (0, E_PAD - E))
    src_r = src.reshape(NC * NS, BATCHES, KB)
    dst_r = dst.reshape(NC * NS, BATCHES, KB)
    w_r = wgt.reshape(NC * NS, BATCHES, KB)

    x_pad = jnp.pad(x, ((0, N_PAD - N_NODES), (0, 0)))

    sc2 = _make_sc_scatter(2, 128)
    sc4 = _make_sc_scatter(4, 128)
    sc1 = _make_sc_scatter(1, 128)

    W4p = jnp.pad(W4, ((0, 0), (0, 64)))
    b4p = jnp.pad(b4, (0, 64))
    Wpp = jnp.pad(Wp, ((0, 64), (0, 0)))
    Wvp = jnp.pad(Wv, ((0, 64), (0, 0)))

    agg1 = sc2(x_pad[:, :128], x_pad[:, 128:], src_r, dst_r, w_r)
    h1 = _tc_matmul(agg1, W1, b1, relu=True)

    agg2 = sc4(h1[0], h1[1], h1[2], h1[3], src_r, dst_r, w_r)
    h2 = _tc_matmul(agg2, W2, b2, relu=True)

    agg3 = sc4(h2[0], h2[1], h2[2], h2[3], src_r, dst_r, w_r)
    t4 = _tc_matmul_fused(agg3, W3, b3, W4p)

    agg4 = sc1(t4, src_r, dst_r, w_r)
    pi_full, v = _tc_readout(agg4, b4p, Wpp, bp, Wvp, bv)
    return pi_full[:N_NODES], v


# speedup vs baseline: 2.1708x; 1.1601x over previous
import functools

import jax
import jax.numpy as jnp
from jax import lax
from jax.experimental import pallas as pl
from jax.experimental.pallas import tpu as pltpu
from jax.experimental.pallas import tpu_sc as plsc

N_NODES = 10000
N_PAD = 10240
E = 160000
NC = 2
NS = 16
KB = 64
E_PER_TILE = 5120
E_PAD = E_PER_TILE * NC * NS
BATCHES = E_PER_TILE // KB
ROWS_PER_TILE = N_PAD // NS


NBUF = 2


@functools.lru_cache(maxsize=None)
def _make_sc_scatter(n_chunks: int, fc: int):
    mesh = plsc.VectorSubcoreMesh(core_axis_name="c", subcore_axis_name="s")
    n_lane = fc // 16

    @functools.partial(
        pl.kernel,
        out_type=jax.ShapeDtypeStruct((n_chunks, NC, N_PAD, fc), jnp.float32),
        mesh=mesh,
        scratch_types=[
            pltpu.VMEM((BATCHES, KB), jnp.int32),
            pltpu.VMEM((BATCHES, KB), jnp.int32),
            pltpu.VMEM((BATCHES, KB), jnp.float32),
            [pltpu.VMEM((KB, fc), jnp.float32)] * NBUF,
            pltpu.VMEM_SHARED((N_PAD, fc), jnp.float32),
            [pltpu.SemaphoreType.DMA] * NBUF,
            [pltpu.SemaphoreType.DMA] * NBUF,
        ],
    )
    def sc_scatter(*refs):
        tables = refs[:n_chunks]
        src_h, dst_h, w_h, out = refs[n_chunks:n_chunks + 4]
        src_v, dst_v, w_v, bufs, acc, gsems, ssems = refs[n_chunks + 4:]

        cid = lax.axis_index("c")
        sid = lax.axis_index("s")
        wid = cid * NS + sid
        pltpu.sync_copy(src_h.at[wid], src_v)
        pltpu.sync_copy(dst_h.at[wid], dst_v)
        pltpu.sync_copy(w_h.at[wid], w_v)

        row0 = sid * ROWS_PER_TILE
        for c in range(n_chunks):
            table = tables[c]

            def start_g(j, p):
                pltpu.async_copy(table.at[src_v.at[j]], bufs[p], gsems[p])

            def wait_g(p):
                pltpu.make_async_copy(table.at[src_v.at[0]], bufs[p],
                                      gsems[p]).wait()

            def start_s(j, p):
                pltpu.async_copy(bufs[p], acc.at[dst_v.at[j]], ssems[p],
                                 add=True)

            def wait_s(p):
                pltpu.make_async_copy(bufs[p], acc.at[dst_v.at[0]],
                                      ssems[p]).wait()

            def scale(p, j):
                buf = bufs[p]

                def grp(g, carry2):
                    wvec = w_v[j, pl.ds(g * 16, 16)]
                    for i2 in range(16):
                        s = wvec[i2]
                        r = g * 16 + i2
                        for l in range(n_lane):
                            buf[r, pl.ds(l * 16, 16)] = (
                                buf[r, pl.ds(l * 16, 16)] * s)
                    return carry2
                lax.fori_loop(0, KB // 16, grp, 0)

            def zrow(i, carry):
                for l in range(n_lane):
                    bufs[0][i, pl.ds(l * 16, 16)] = jnp.zeros((16,), jnp.float32)
                return carry
            lax.fori_loop(0, KB, zrow, 0)

            def zblk(b, carry):
                pltpu.sync_copy(bufs[0], acc.at[pl.ds(row0 + b * KB, KB)])
                return carry
            lax.fori_loop(0, ROWS_PER_TILE // KB, zblk, 0)
            plsc.subcore_barrier()

            start_g(0, 0)

            def group(jj, carry):
                j0 = jj * NBUF
                for p in range(NBUF):
                    j = j0 + p
                    pn = (p + 1) % NBUF
                    wait_g(p)
                    @pl.when(j >= NBUF - 1)
                    def _():
                        wait_s(pn)

                    @pl.when(j + 1 < BATCHES)
                    def _():
                        start_g(j + 1, pn)
                    scale(p, j)
                    start_s(j, p)
                return carry
            lax.fori_loop(0, BATCHES // NBUF, group, 0)
            for i in range(1, NBUF):
                wait_s((BATCHES - NBUF + i) % NBUF)
            plsc.subcore_barrier()

            pltpu.sync_copy(acc.at[pl.ds(row0, ROWS_PER_TILE)],
                            out.at[c, cid, pl.ds(row0, ROWS_PER_TILE)])

    return sc_scatter


def _mm_body(p_ref, w_ref, b_ref, o_ref, acc_ref, *, n_ci, relu):
    ci = pl.program_id(2)

    @pl.when(ci == 0)
    def _():
        acc_ref[...] = jnp.zeros_like(acc_ref)

    a = p_ref[0, 0] + p_ref[0, 1]
    acc_ref[...] += jnp.dot(a, w_ref[...], preferred_element_type=jnp.float32)

    @pl.when(ci == n_ci - 1)
    def _():
        r = acc_ref[...] + b_ref[...]
        if relu:
            r = jnp.maximum(r, 0.0)
        o_ref[0] = r


def _tc_matmul(p, w, b, relu, nb=1024):
    n_ci, _, n_pad, fc = p.shape
    n_co = w.shape[1] // fc
    grid = (n_pad // nb, n_co, n_ci)
    return pl.pallas_call(
        functools.partial(_mm_body, n_ci=n_ci, relu=relu),
        grid=grid,
        in_specs=[
            pl.BlockSpec((1, 2, nb, fc), lambda i, co, ci: (ci, 0, i, 0)),
            pl.BlockSpec((fc, fc), lambda i, co, ci: (ci, co)),
            pl.BlockSpec((1, fc), lambda i, co, ci: (0, co)),
        ],
        out_specs=pl.BlockSpec((1, nb, fc), lambda i, co, ci: (co, i, 0)),
        out_shape=jax.ShapeDtypeStruct((n_co, n_pad, fc), jnp.float32),
        scratch_shapes=[pltpu.VMEM((nb, fc), jnp.float32)],
    )(p, w, b.reshape(1, -1))


def _mm_fused_body(p_ref, w3_ref, b3_ref, w4_ref, o_ref, acc_ref, *, n_ci):
    ci = pl.program_id(1)

    @pl.when(ci == 0)
    def _():
        acc_ref[...] = jnp.zeros_like(acc_ref)

    a = p_ref[0, 0] + p_ref[0, 1]
    acc_ref[...] += jnp.dot(a, w3_ref[...], preferred_element_type=jnp.float32)

    @pl.when(ci == n_ci - 1)
    def _():
        h = jnp.maximum(acc_ref[...] + b3_ref[...], 0.0)
        o_ref[...] = jnp.dot(h, w4_ref[...], preferred_element_type=jnp.float32)


def _tc_matmul_fused(p, w3, b3, w4, nb=1024):
    n_ci, _, n_pad, fc = p.shape
    h = w3.shape[1]
    f_out = w4.shape[1]
    grid = (n_pad // nb, n_ci)
    return pl.pallas_call(
        functools.partial(_mm_fused_body, n_ci=n_ci),
        grid=grid,
        in_specs=[
            pl.BlockSpec((1, 2, nb, fc), lambda i, ci: (ci, 0, i, 0)),
            pl.BlockSpec((fc, h), lambda i, ci: (ci, 0)),
            pl.BlockSpec((1, h), lambda i, ci: (0, 0)),
            pl.BlockSpec((h, f_out), lambda i, ci: (0, 0)),
        ],
        out_specs=pl.BlockSpec((nb, f_out), lambda i, ci: (i, 0)),
        out_shape=jax.ShapeDtypeStruct((n_pad, f_out), jnp.float32),
        scratch_shapes=[pltpu.VMEM((nb, h), jnp.float32)],
    )(p, w3, b3.reshape(1, -1), w4)


def _readout_body(p_ref, b4_ref, wp_ref, bp_ref, wv_ref, bv_ref,
                  pi_ref, v_ref, cs_ref, *, nb, n_blocks):
    i = pl.program_id(0)
    h = p_ref[0, 0] + p_ref[0, 1] + b4_ref[...]
    rows = i * nb + lax.broadcasted_iota(jnp.int32, (nb, 1), 0)
    hm = jnp.where(rows < N_NODES, h, 0.0)
    pi_ref[...] = jnp.dot(hm, wp_ref[...], preferred_element_type=jnp.float32) + bp_ref[...]

    @pl.when(i == 0)
    def _():
        cs_ref[...] = jnp.zeros_like(cs_ref)

    cs_ref[...] += jnp.sum(hm, axis=0, keepdims=True)

    @pl.when(i == n_blocks - 1)
    def _():
        v_ref[...] = jnp.dot(cs_ref[...], wv_ref[...],
                             preferred_element_type=jnp.float32) + bv_ref[...]


def _tc_readout(p, b4, wp, bp, wv, bv, nb=1024):
    _, _, n_pad, fc = p.shape
    n_blocks = n_pad // nb
    return pl.pallas_call(
        functools.partial(_readout_body, nb=nb, n_blocks=n_blocks),
        grid=(n_blocks,),
        in_specs=[
            pl.BlockSpec((1, 2, nb, fc), lambda i: (0, 0, i, 0)),
            pl.BlockSpec((1, fc), lambda i: (0, 0)),
            pl.BlockSpec((fc, 1), lambda i: (0, 0)),
            pl.BlockSpec((1, 1), lambda i: (0, 0)),
            pl.BlockSpec((fc, 1), lambda i: (0, 0)),
            pl.BlockSpec((1, 1), lambda i: (0, 0)),
        ],
        out_specs=[
            pl.BlockSpec((nb, 1), lambda i: (i, 0)),
            pl.BlockSpec((1, 1), lambda i: (0, 0)),
        ],
        out_shape=[
            jax.ShapeDtypeStruct((n_pad, 1), jnp.float32),
            jax.ShapeDtypeStruct((1, 1), jnp.float32),
        ],
        scratch_shapes=[pltpu.VMEM((1, fc), jnp.float32)],
    )(p, b4.reshape(1, -1), wp, bp.reshape(1, 1), wv, bv.reshape(1, 1))


def kernel(x, edge_index, w, W1, b1, W2, b2, W3, b3, W4, b4, Wp, bp, Wv, bv):
    src = jnp.pad(edge_index[0].astype(jnp.int32), (0, E_PAD - E))
    dst = jnp.pad(edge_index[1].astype(jnp.int32), (0, E_PAD - E))
    wgt = jnp.pad(w.astype(jnp.float32), (0, E_PAD - E))
    src_r = src.reshape(NC * NS, BATCHES, KB)
    dst_r = dst.reshape(NC * NS, BATCHES, KB)
    w_r = wgt.reshape(NC * NS, BATCHES, KB)

    x_pad = jnp.pad(x, ((0, N_PAD - N_NODES), (0, 0)))

    sc2 = _make_sc_scatter(2, 128)
    sc4 = _make_sc_scatter(4, 128)
    sc1 = _make_sc_scatter(1, 128)

    W4p = jnp.pad(W4, ((0, 0), (0, 64)))
    b4p = jnp.pad(b4, (0, 64))
    Wpp = jnp.pad(Wp, ((0, 64), (0, 0)))
    Wvp = jnp.pad(Wv, ((0, 64), (0, 0)))

    agg1 = sc2(x_pad[:, :128], x_pad[:, 128:], src_r, dst_r, w_r)
    h1 = _tc_matmul(agg1, W1, b1, relu=True)

    agg2 = sc4(h1[0], h1[1], h1[2], h1[3], src_r, dst_r, w_r)
    h2 = _tc_matmul(agg2, W2, b2, relu=True)

    agg3 = sc4(h2[0], h2[1], h2[2], h2[3], src_r, dst_r, w_r)
    t4 = _tc_matmul_fused(agg3, W3, b3, W4p)

    agg4 = sc1(t4, src_r, dst_r, w_r)
    pi_full, v = _tc_readout(agg4, b4p, Wpp, bp, Wvp, bv)
    return pi_full[:N_NODES], v


# speedup vs baseline: 5.3197x; 2.4505x over previous
import functools

import jax
import jax.numpy as jnp
from jax import lax
from jax.experimental import pallas as pl
from jax.experimental.pallas import tpu as pltpu
from jax.experimental.pallas import tpu_sc as plsc

N_NODES = 10000
N_PAD = 10240
E = 160000
NC = 2
NS = 16
KB = 80
E_PER_TILE = 5120
E_PAD = E_PER_TILE * NC * NS
BATCHES = E_PER_TILE // KB
ROWS_PER_TILE = N_PAD // NS
ZB = KB


RING = 4


@functools.lru_cache(maxsize=None)
def _make_sc_scatter(n_chunks: int, fc: int):
    mesh = plsc.VectorSubcoreMesh(core_axis_name="c", subcore_axis_name="s")
    n_lane = fc // 16

    @functools.partial(
        pl.kernel,
        out_type=jax.ShapeDtypeStruct((n_chunks, NC, N_PAD, fc), jnp.float32),
        mesh=mesh,
        scratch_types=[
            [pltpu.VMEM((KB, fc), jnp.float32)] * RING,
            [pltpu.VMEM((3, KB), jnp.int32)] * RING,
            pltpu.VMEM_SHARED((N_PAD, fc), jnp.float32),
            [pltpu.SemaphoreType.DMA] * RING,
            [pltpu.SemaphoreType.DMA] * RING,
            [pltpu.SemaphoreType.DMA] * RING,
        ],
        compiler_params=pltpu.CompilerParams(needs_layout_passes=False),
    )
    def sc_scatter(*refs):
        tables = refs[:n_chunks]
        pk, out = refs[n_chunks:n_chunks + 2]
        bufs, pbufs, acc, gsems, ssems, psems = refs[n_chunks + 2:]

        cid = lax.axis_index("c")
        sid = lax.axis_index("s")
        wid = cid * NS + sid
        row0 = sid * ROWS_PER_TILE

        for c in range(n_chunks):
            table = tables[c]

            def start_p(j, p):
                pltpu.async_copy(pk.at[wid, j], pbufs[p], psems[p])

            def wait_p(p):
                pltpu.make_async_copy(pk.at[wid, 0], pbufs[p], psems[p]).wait()

            def start_g(p):
                pltpu.async_copy(table.at[pbufs[p].at[0]], bufs[p], gsems[p])

            def wait_g(p):
                pltpu.make_async_copy(table.at[pbufs[p].at[0]], bufs[p],
                                      gsems[p]).wait()

            def start_s(p):
                pltpu.async_copy(bufs[p], acc.at[pbufs[p].at[1]], ssems[p],
                                 add=True)

            def wait_s(p):
                pltpu.make_async_copy(bufs[p], acc.at[pbufs[p].at[1]],
                                      ssems[p]).wait()

            def scale(p):
                buf = bufs[p]

                def grp(g, carry2):
                    wvec = plsc.bitcast(pbufs[p][2, pl.ds(g * 16, 16)],
                                        jnp.float32)
                    for i2 in range(16):
                        s = wvec[i2]
                        r = g * 16 + i2
                        for l in range(n_lane):
                            buf[r, pl.ds(l * 16, 16)] = (
                                buf[r, pl.ds(l * 16, 16)] * s)
                    return carry2
                lax.fori_loop(0, KB // 16, grp, 0)

            def zrow(i, carry):
                for l in range(n_lane):
                    bufs[0][i, pl.ds(l * 16, 16)] = jnp.zeros((16,), jnp.float32)
                return carry
            lax.fori_loop(0, ZB, zrow, 0)

            def zblk(b, carry):
                pltpu.sync_copy(bufs[0], acc.at[pl.ds(row0 + b * ZB, ZB)])
                return carry
            lax.fori_loop(0, ROWS_PER_TILE // ZB, zblk, 0)
            plsc.subcore_barrier()

            start_p(0, 0)
            start_p(1, 1)
            wait_p(0)
            start_g(0)

            def group(jj, carry):
                j0 = jj * RING
                for p in range(RING):
                    j = j0 + p
                    p2 = (p + 2) % RING
                    p1 = (p + 1) % RING
                    wait_g(p)
                    @pl.when((j >= 2) & (j + 2 < BATCHES))
                    def _():
                        wait_s(p2)

                    @pl.when(j + 2 < BATCHES)
                    def _():
                        start_p(j + 2, p2)

                    @pl.when(j + 1 < BATCHES)
                    def _():
                        wait_p(p1)
                        start_g(p1)
                    scale(p)
                    start_s(p)
                return carry
            lax.fori_loop(0, BATCHES // RING, group, 0)
            for p in range(RING):
                wait_s(p)
            plsc.subcore_barrier()

            pltpu.sync_copy(acc.at[pl.ds(row0, ROWS_PER_TILE)],
                            out.at[c, cid, pl.ds(row0, ROWS_PER_TILE)])

    return sc_scatter


def _mm_body(p_ref, w_ref, b_ref, o_ref, acc_ref, *, n_ci, relu):
    ci = pl.program_id(2)

    @pl.when(ci == 0)
    def _():
        acc_ref[...] = jnp.zeros_like(acc_ref)

    a = p_ref[0, 0] + p_ref[0, 1]
    acc_ref[...] += jnp.dot(a, w_ref[...], preferred_element_type=jnp.float32)

    @pl.when(ci == n_ci - 1)
    def _():
        r = acc_ref[...] + b_ref[...]
        if relu:
            r = jnp.maximum(r, 0.0)
        o_ref[0] = r


def _tc_matmul(p, w, b, relu, nb=1024):
    n_ci, _, n_pad, fc = p.shape
    n_co = w.shape[1] // fc
    grid = (n_pad // nb, n_co, n_ci)
    return pl.pallas_call(
        functools.partial(_mm_body, n_ci=n_ci, relu=relu),
        grid=grid,
        in_specs=[
            pl.BlockSpec((1, 2, nb, fc), lambda i, co, ci: (ci, 0, i, 0)),
            pl.BlockSpec((fc, fc), lambda i, co, ci: (ci, co)),
            pl.BlockSpec((1, fc), lambda i, co, ci: (0, co)),
        ],
        out_specs=pl.BlockSpec((1, nb, fc), lambda i, co, ci: (co, i, 0)),
        out_shape=jax.ShapeDtypeStruct((n_co, n_pad, fc), jnp.float32),
        scratch_shapes=[pltpu.VMEM((nb, fc), jnp.float32)],
    )(p, w, b.reshape(1, -1))


def _mm_fused_body(p_ref, w3_ref, b3_ref, w4_ref, o_ref, acc_ref, *, n_ci):
    ci = pl.program_id(1)

    @pl.when(ci == 0)
    def _():
        acc_ref[...] = jnp.zeros_like(acc_ref)

    a = p_ref[0, 0] + p_ref[0, 1]
    acc_ref[...] += jnp.dot(a, w3_ref[...], preferred_element_type=jnp.float32)

    @pl.when(ci == n_ci - 1)
    def _():
        h = jnp.maximum(acc_ref[...] + b3_ref[...], 0.0)
        o_ref[...] = jnp.dot(h, w4_ref[...], preferred_element_type=jnp.float32)


def _tc_matmul_fused(p, w3, b3, w4, nb=1024):
    n_ci, _, n_pad, fc = p.shape
    h = w3.shape[1]
    f_out = w4.shape[1]
    grid = (n_pad // nb, n_ci)
    return pl.pallas_call(
        functools.partial(_mm_fused_body, n_ci=n_ci),
        grid=grid,
        in_specs=[
            pl.BlockSpec((1, 2, nb, fc), lambda i, ci: (ci, 0, i, 0)),
            pl.BlockSpec((fc, h), lambda i, ci: (ci, 0)),
            pl.BlockSpec((1, h), lambda i, ci: (0, 0)),
            pl.BlockSpec((h, f_out), lambda i, ci: (0, 0)),
        ],
        out_specs=pl.BlockSpec((nb, f_out), lambda i, ci: (i, 0)),
        out_shape=jax.ShapeDtypeStruct((n_pad, f_out), jnp.float32),
        scratch_shapes=[pltpu.VMEM((nb, h), jnp.float32)],
    )(p, w3, b3.reshape(1, -1), w4)


def _readout_body(p_ref, b4_ref, wp_ref, bp_ref, wv_ref, bv_ref,
                  pi_ref, v_ref, cs_ref, *, nb, n_blocks):
    i = pl.program_id(0)
    h = p_ref[0, 0] + p_ref[0, 1] + b4_ref[...]
    rows = i * nb + lax.broadcasted_iota(jnp.int32, (nb, 1), 0)
    hm = jnp.where(rows < N_NODES, h, 0.0)
    pi_ref[...] = jnp.dot(hm, wp_ref[...], preferred_element_type=jnp.float32) + bp_ref[...]

    @pl.when(i == 0)
    def _():
        cs_ref[...] = jnp.zeros_like(cs_ref)

    cs_ref[...] += jnp.sum(hm, axis=0, keepdims=True)

    @pl.when(i == n_blocks - 1)
    def _():
        v_ref[...] = jnp.dot(cs_ref[...], wv_ref[...],
                             preferred_element_type=jnp.float32) + bv_ref[...]


def _tc_readout(p, b4, wp, bp, wv, bv, nb=1024):
    _, _, n_pad, fc = p.shape
    n_blocks = n_pad // nb
    return pl.pallas_call(
        functools.partial(_readout_body, nb=nb, n_blocks=n_blocks),
        grid=(n_blocks,),
        in_specs=[
            pl.BlockSpec((1, 2, nb, fc), lambda i: (0, 0, i, 0)),
            pl.BlockSpec((1, fc), lambda i: (0, 0)),
            pl.BlockSpec((fc, 1), lambda i: (0, 0)),
            pl.BlockSpec((1, 1), lambda i: (0, 0)),
            pl.BlockSpec((fc, 1), lambda i: (0, 0)),
            pl.BlockSpec((1, 1), lambda i: (0, 0)),
        ],
        out_specs=[
            pl.BlockSpec((nb, 1), lambda i: (i, 0)),
            pl.BlockSpec((1, 1), lambda i: (0, 0)),
        ],
        out_shape=[
            jax.ShapeDtypeStruct((n_pad, 1), jnp.float32),
            jax.ShapeDtypeStruct((1, 1), jnp.float32),
        ],
        scratch_shapes=[pltpu.VMEM((1, fc), jnp.float32)],
    )(p, b4.reshape(1, -1), wp, bp.reshape(1, 1), wv, bv.reshape(1, 1))


def kernel(x, edge_index, w, W1, b1, W2, b2, W3, b3, W4, b4, Wp, bp, Wv, bv):
    pad_idx = jnp.arange(E_PAD - E, dtype=jnp.int32) % N_NODES
    src = jnp.concatenate([edge_index[0].astype(jnp.int32), pad_idx])
    dst = jnp.concatenate([edge_index[1].astype(jnp.int32), pad_idx])
    wgt = jnp.pad(w.astype(jnp.float32), (0, E_PAD - E))
    src_r = src.reshape(NC * NS, BATCHES, KB)
    dst_r = dst.reshape(NC * NS, BATCHES, KB)
    wb_r = lax.bitcast_convert_type(wgt, jnp.int32).reshape(NC * NS, BATCHES, KB)
    pk = jnp.stack([src_r, dst_r, wb_r], axis=2)

    x_pad = jnp.pad(x, ((0, N_PAD - N_NODES), (0, 0)))

    sc2 = _make_sc_scatter(2, 128)
    sc4 = _make_sc_scatter(4, 128)
    sc1 = _make_sc_scatter(1, 128)

    W4p = jnp.pad(W4, ((0, 0), (0, 64)))
    b4p = jnp.pad(b4, (0, 64))
    Wpp = jnp.pad(Wp, ((0, 64), (0, 0)))
    Wvp = jnp.pad(Wv, ((0, 64), (0, 0)))

    agg1 = sc2(x_pad[:, :128], x_pad[:, 128:], pk)
    h1 = _tc_matmul(agg1, W1, b1, relu=True)

    agg2 = sc4(h1[0], h1[1], h1[2], h1[3], pk)
    h2 = _tc_matmul(agg2, W2, b2, relu=True)

    agg3 = sc4(h2[0], h2[1], h2[2], h2[3], pk)
    t4 = _tc_matmul_fused(agg3, W3, b3, W4p)

    agg4 = sc1(t4, pk)
    pi_full, v = _tc_readout(agg4, b4p, Wpp, bp, Wvp, bv)
    return pi_full[:N_NODES], v
